# baseline (speedup 1.0000x reference)
import functools

import jax
import jax.numpy as jnp
from jax import lax
from jax._src import core as _jcore
from jax.experimental import compute_on
from jax.experimental import pallas as pl
from jax.experimental.pallas import tpu as pltpu
from jax.experimental.pallas import tpu_sc as plsc

_N = 10000
_E = 320000
_D = 128
_H = 128
_S = 5
_G = 256

_NC = 2
_NS = 16
_NW = _NC * _NS
_EPW = _E // _NW
_CH = 40
_NCHUNK = _EPW // _CH
_CHP = 80
_EPS = _E // _NS
_NCHP = _EPS // _CHP
_NG = 10
_CHG = _NCHP // _NG
_RING = 3
_RCH = 80
_NRCH = _N // _RCH

_R = 1000
_NB = _N // _R

_sc_mesh = plsc.VectorSubcoreMesh(core_axis_name="c", subcore_axis_name="s")
_sc_mesh1 = plsc.VectorSubcoreMesh(core_axis_name="c", subcore_axis_name="s",
                                   num_cores=1)


def _sc_offload(f):
    return compute_on.compute_on2(
        f, compute_type="tpu_sparsecore",
        out_memory_spaces=_jcore.MemorySpace.Device)


@functools.partial(
    pl.kernel,
    out_type=jax.ShapeDtypeStruct((_NC, _N, 16), jnp.float32),
    mesh=_sc_mesh,
    scratch_types=[
        pltpu.VMEM((_NCHUNK, _CH), jnp.int32),
        pltpu.VMEM((_CH, 16), jnp.float32),
        pltpu.VMEM((_RCH, 16), jnp.float32),
        pltpu.VMEM_SHARED((_N, 16), jnp.float32),
    ],
)
def _deg_kernel(dst_hbm, out_hbm, didx, ones_v, zbuf, acc):
    cid = lax.axis_index("c")
    sid = lax.axis_index("s")
    w = cid * _NS + sid

    @pl.loop(0, _RCH)
    def _zero(i):
        zbuf[i] = jnp.zeros((16,), jnp.float32)

    @pl.loop(0, _CH)
    def _one(i):
        ones_v[i] = jnp.full((16,), 1.0, jnp.float32)

    @pl.loop(sid, _NRCH, step=_NS)
    def _init(r):
        pltpu.sync_copy(zbuf, acc.at[pl.ds(r * _RCH, _RCH)])

    plsc.subcore_barrier()

    pltpu.sync_copy(dst_hbm.at[w], didx)

    @pl.loop(0, _NCHUNK)
    def _scat(j):
        pltpu.sync_copy(ones_v, acc.at[didx.at[j]], add=True)

    plsc.subcore_barrier()

    @pl.loop(sid, _NRCH, step=_NS)
    def _out(r):
        pltpu.sync_copy(acc.at[pl.ds(r * _RCH, _RCH)],
                        out_hbm.at[cid, pl.ds(r * _RCH, _RCH)])


@functools.partial(
    pl.kernel,
    out_type=jax.ShapeDtypeStruct((_N, _H), jnp.float32),
    mesh=_sc_mesh1,
    scratch_types=[
        pltpu.VMEM((2, _CHG, _CHP), jnp.int32),
        pltpu.VMEM((2, _CHG, _CHP), jnp.int32),
        pltpu.VMEM((_RING, _CHP, _H), jnp.float32),
        pltpu.VMEM_SHARED((_N, _H), jnp.float32),
    ] + [pltpu.SemaphoreType.DMA] * (_RING + 2),
)
def _pass_kernel(src_hbm, dst_hbm, table_hbm, out_hbm,
                 sidx, didx, rows, acc, *sems):
    sid = lax.axis_index("s")

    @pl.loop(0, _RCH)
    def _zero(i):
        for k in range(_H // 16):
            rows[0, i, pl.ds(k * 16, 16)] = jnp.zeros((16,), jnp.float32)

    @pl.loop(sid, _NRCH, step=_NS)
    def _init(r):
        pltpu.sync_copy(rows.at[0], acc.at[pl.ds(r * _RCH, _RCH)])

    plsc.subcore_barrier()

    def start(sl, j, b):
        pltpu.make_async_copy(table_hbm.at[sidx.at[sl, j]], rows.at[b],
                              sems[b]).start()

    def wait(b):
        pltpu.make_async_copy(table_hbm.at[sidx.at[0, 0]], rows.at[b],
                              sems[b]).wait()

    def idx_start(g, sl):
        pltpu.make_async_copy(src_hbm.at[sid, g], sidx.at[sl],
                              sems[_RING + sl]).start()
        pltpu.make_async_copy(dst_hbm.at[sid, g], didx.at[sl],
                              sems[_RING + sl]).start()

    def idx_wait(sl):
        pltpu.make_async_copy(src_hbm.at[sid, 0], sidx.at[sl],
                              sems[_RING + sl]).wait()
        pltpu.make_async_copy(dst_hbm.at[sid, 0], didx.at[sl],
                              sems[_RING + sl]).wait()

    def scat(sl, j, b):
        pltpu.sync_copy(rows.at[b], acc.at[didx.at[sl, j]], add=True)

    R = _RING
    bulk = (_CHG - R) // R
    rem = _CHG - R * bulk - R
    idx_start(0, 0)
    for g in range(_NG):
        sl = g % 2
        idx_wait(sl)
        if g + 1 < _NG:
            idx_start(g + 1, (g + 1) % 2)
        for b in range(R):
            start(sl, b, b)

        @pl.loop(0, bulk)
        def _main(p):
            j = R * p
            for b in range(R):
                wait(b)
                scat(sl, j + b, b)
                start(sl, j + b + R, b)

        j0 = R * bulk
        for i in range(rem):
            b = i % R
            wait(b)
            scat(sl, j0 + i, b)
            start(sl, j0 + R + i, b)
        for i in range(rem, rem + R):
            b = i % R
            wait(b)
            scat(sl, j0 + i, b)

    plsc.subcore_barrier()

    @pl.loop(sid, _NRCH, step=_NS)
    def _out(r):
        pltpu.sync_copy(acc.at[pl.ds(r * _RCH, _RCH)],
                        out_hbm.at[pl.ds(r * _RCH, _RCH)])


_NT = (((1,), (1,)), ((), ()))


def _prep_body(deg_ref, x_ref, xn_ref, nb_ref):
    deg = deg_ref[0] + deg_ref[1]
    norm = lax.rsqrt(jnp.maximum(deg[:, :1], 1.0))
    nb = jnp.broadcast_to(norm, (_R, _H))
    nb_ref[...] = nb
    xn_ref[...] = x_ref[...] * nb


_prep_call = pl.pallas_call(
    _prep_body,
    grid=(_NB,),
    in_specs=[
        pl.BlockSpec((_NC, _R, 16), lambda i: (0, i, 0)),
        pl.BlockSpec((_R, _D), lambda i: (i, 0)),
    ],
    out_specs=[
        pl.BlockSpec((_R, _D), lambda i: (i, 0)),
        pl.BlockSpec((_R, _H), lambda i: (i, 0)),
    ],
    out_shape=[
        jax.ShapeDtypeStruct((_N, _D), jnp.float32),
        jax.ShapeDtypeStruct((_N, _H), jnp.float32),
    ],
)


def _combine_body(y_ref, nb_ref, z_ref):
    z_ref[...] = y_ref[...] * nb_ref[...]


_combine_call = pl.pallas_call(
    _combine_body,
    grid=(_NB,),
    in_specs=[
        pl.BlockSpec((_R, _H), lambda i: (i, 0)),
        pl.BlockSpec((_R, _H), lambda i: (i, 0)),
    ],
    out_specs=pl.BlockSpec((_R, _H), lambda i: (i, 0)),
    out_shape=jax.ShapeDtypeStruct((_N, _H), jnp.float32),
)


def _b1_body(z_ref, p_ref, w_ref, t_ref, ssum_ref, ssq_ref):
    i = pl.program_id(1)
    z = z_ref[...]
    zp = lax.dot_general(z, p_ref[0], _NT, preferred_element_type=jnp.float32)
    t = lax.dot_general(zp, w_ref[0], _NT, preferred_element_type=jnp.float32)
    t_ref[0] = t

    @pl.when(i == 0)
    def _():
        ssum_ref[0] = jnp.zeros((1, _H), jnp.float32)
        ssq_ref[0] = jnp.zeros((1, _H), jnp.float32)

    ssum_ref[0] += jnp.sum(t, axis=0, keepdims=True)
    ssq_ref[0] += jnp.sum(t * t, axis=0, keepdims=True)


_b1_call = pl.pallas_call(
    _b1_body,
    grid=(_S, _NB),
    in_specs=[
        pl.BlockSpec((_R, _H), lambda s, i: (i, 0)),
        pl.BlockSpec((1, _H, _D), lambda s, i: (s, 0, 0)),
        pl.BlockSpec((1, _H, _H), lambda s, i: (s, 0, 0)),
    ],
    out_specs=[
        pl.BlockSpec((1, _R, _H), lambda s, i: (s, i, 0)),
        pl.BlockSpec((1, 1, _H), lambda s, i: (s, 0, 0)),
        pl.BlockSpec((1, 1, _H), lambda s, i: (s, 0, 0)),
    ],
    out_shape=[
        jax.ShapeDtypeStruct((_S, _N, _H), jnp.float32),
        jax.ShapeDtypeStruct((_S, 1, _H), jnp.float32),
        jax.ShapeDtypeStruct((_S, 1, _H), jnp.float32),
    ],
)


def _bn_coeffs(ssum, ssq, gamma, beta):
    mean = ssum * (1.0 / _N)
    var = ssq * (1.0 / _N) - mean * mean
    a = gamma * lax.rsqrt(var + 1e-5)
    b = beta - mean * a
    return a, b


def _b2_body(t_ref, ssum_ref, ssq_ref, g_ref, b_ref, nb_ref, hn_ref):
    a, b = _bn_coeffs(ssum_ref[0], ssq_ref[0], g_ref[0], b_ref[0])
    hn_ref[0] = jnp.maximum(t_ref[0] * a + b, 0.0) * nb_ref[...]


_b2_call = pl.pallas_call(
    _b2_body,
    grid=(_S, _NB),
    in_specs=[
        pl.BlockSpec((1, _R, _H), lambda s, i: (s, i, 0)),
        pl.BlockSpec((1, 1, _H), lambda s, i: (s, 0, 0)),
        pl.BlockSpec((1, 1, _H), lambda s, i: (s, 0, 0)),
        pl.BlockSpec((1, 1, _H), lambda s, i: (s, 0, 0)),
        pl.BlockSpec((1, 1, _H), lambda s, i: (s, 0, 0)),
        pl.BlockSpec((_R, _H), lambda s, i: (i, 0)),
    ],
    out_specs=pl.BlockSpec((1, _R, _H), lambda s, i: (s, i, 0)),
    out_shape=jax.ShapeDtypeStruct((_S, _N, _H), jnp.float32),
)


def _c1_body(z1_ref, w_ref, u_ref, csum_ref, csq_ref):
    i = pl.program_id(1)
    u = lax.dot_general(z1_ref[0], w_ref[0], _NT,
                        preferred_element_type=jnp.float32)
    u_ref[0] = u

    @pl.when(i == 0)
    def _():
        csum_ref[0] = jnp.zeros((1, _H), jnp.float32)
        csq_ref[0] = jnp.zeros((1, _H), jnp.float32)

    csum_ref[0] += jnp.sum(u, axis=0, keepdims=True)
    csq_ref[0] += jnp.sum(u * u, axis=0, keepdims=True)


_c1_call = pl.pallas_call(
    _c1_body,
    grid=(_S, _NB),
    in_specs=[
        pl.BlockSpec((1, _R, _H), lambda s, i: (s, i, 0)),
        pl.BlockSpec((1, _H, _H), lambda s, i: (s, 0, 0)),
    ],
    out_specs=[
        pl.BlockSpec((1, _R, _H), lambda s, i: (s, i, 0)),
        pl.BlockSpec((1, 1, _H), lambda s, i: (s, 0, 0)),
        pl.BlockSpec((1, 1, _H), lambda s, i: (s, 0, 0)),
    ],
    out_shape=[
        jax.ShapeDtypeStruct((_S, _N, _H), jnp.float32),
        jax.ShapeDtypeStruct((_S, 1, _H), jnp.float32),
        jax.ShapeDtypeStruct((_S, 1, _H), jnp.float32),
    ],
)


def _pool_body(u_ref, csum_ref, csq_ref, g_ref, b_ref, batch_ref,
               pooled_ref, cnt_ref):
    i = pl.program_id(0)
    havg = jnp.zeros((_R, _H), jnp.float32)
    for s in range(_S):
        a, b = _bn_coeffs(csum_ref[s], csq_ref[s], g_ref[s], b_ref[s])
        havg = havg + jnp.maximum(u_ref[s] * a + b, 0.0)
    havg = havg * (1.0 / _S)

    bvals = batch_ref[0]
    rows = lax.broadcasted_iota(jnp.int32, (_G, _R), 0)
    oh = (bvals == rows).astype(jnp.float32)

    @pl.when(i == 0)
    def _():
        pooled_ref[...] = jnp.zeros((_G, _H), jnp.float32)
        cnt_ref[...] = jnp.zeros((_G, 1), jnp.float32)

    pooled_ref[...] += jnp.dot(oh, havg, preferred_element_type=jnp.float32)
    cnt_ref[...] += jnp.sum(oh, axis=1, keepdims=True)


_pool_call = pl.pallas_call(
    _pool_body,
    grid=(_NB,),
    in_specs=[
        pl.BlockSpec((_S, _R, _H), lambda i: (0, i, 0)),
        pl.BlockSpec((_S, 1, _H), lambda i: (0, 0, 0)),
        pl.BlockSpec((_S, 1, _H), lambda i: (0, 0, 0)),
        pl.BlockSpec((_S, 1, _H), lambda i: (0, 0, 0)),
        pl.BlockSpec((_S, 1, _H), lambda i: (0, 0, 0)),
        pl.BlockSpec((1, 1, _R), lambda i: (i, 0, 0)),
    ],
    out_specs=[
        pl.BlockSpec((_G, _H), lambda i: (0, 0)),
        pl.BlockSpec((_G, 1), lambda i: (0, 0)),
    ],
    out_shape=[
        jax.ShapeDtypeStruct((_G, _H), jnp.float32),
        jax.ShapeDtypeStruct((_G, 1), jnp.float32),
    ],
)


def _head_body(pooled_ref, cnt_ref, fc1w_ref, fc1b_ref, outw_ref, outb_ref,
               out_ref):
    y = pooled_ref[...] / jnp.maximum(cnt_ref[...], 1.0)
    nrm = jnp.sqrt(jnp.sum(y * y, axis=1, keepdims=True))
    h = y / jnp.maximum(nrm, 1e-12)
    h1 = lax.dot_general(h, fc1w_ref[...], _NT,
                         preferred_element_type=jnp.float32) + fc1b_ref[...]
    h1 = jnp.where(h1 >= 0, h1, 0.01 * h1)
    o = jnp.sum(h1 * outw_ref[...], axis=1, keepdims=True) + outb_ref[...]
    out_ref[...] = 1.0 / (1.0 + jnp.exp(-o))


_head_call = pl.pallas_call(
    _head_body,
    out_shape=jax.ShapeDtypeStruct((_G, 1), jnp.float32),
)


def kernel(x, fc_proj_W, gcn_W, bn_gamma, bn_beta, fc1_W, fc1_b, out_W,
           out_b, edge_index, batch):
    src = edge_index[0].reshape(_NS, _NG, _CHG, _CHP)
    dst = edge_index[1].reshape(_NS, _NG, _CHG, _CHP)

    deg2 = _sc_offload(_deg_kernel)(edge_index[1].reshape(_NW, _NCHUNK, _CH))
    xn, normb = _prep_call(deg2, x)

    y2 = _sc_offload(_pass_kernel)(src, dst, xn)
    z = _combine_call(y2, normb)

    t, ssum, ssq = _b1_call(z, fc_proj_W, gcn_W[:, 0])
    hn = _b2_call(t, ssum, ssq,
                  bn_gamma[:, 0].reshape(_S, 1, _H),
                  bn_beta[:, 0].reshape(_S, 1, _H), normb)

    z1 = jnp.stack(
        [_combine_call(_sc_offload(_pass_kernel)(src, dst, hn[s]), normb)
         for s in range(_S)])

    u, csum, csq = _c1_call(z1, gcn_W[:, 1])
    pooled, cnt = _pool_call(u, csum, csq,
                             bn_gamma[:, 1].reshape(_S, 1, _H),
                             bn_beta[:, 1].reshape(_S, 1, _H),
                             batch.reshape(_NB, 1, _R))
    return _head_call(pooled, cnt, fc1_W, fc1_b.reshape(1, _H), out_W,
                      out_b.reshape(1, 1))


# speedup vs baseline: 1.0383x; 1.0383x over previous
import functools

import jax
import jax.numpy as jnp
from jax import lax
from jax._src import core as _jcore
from jax.experimental import compute_on
from jax.experimental import pallas as pl
from jax.experimental.pallas import tpu as pltpu
from jax.experimental.pallas import tpu_sc as plsc

_N = 10000
_E = 320000
_D = 128
_H = 128
_S = 5
_G = 256

_NC = 2
_NS = 16
_NW = _NC * _NS
_EPW = _E // _NW
_CH = 40
_NCHUNK = _EPW // _CH
_CHP = 80
_EPS = _E // _NS
_NCHP = _EPS // _CHP
_NG = 5
_CHG = _NCHP // _NG
_RING = 3
_RCH = 80
_NRCH = _N // _RCH

_R = 1000
_NB = _N // _R

_sc_mesh = plsc.VectorSubcoreMesh(core_axis_name="c", subcore_axis_name="s")
_sc_mesh1 = plsc.VectorSubcoreMesh(core_axis_name="c", subcore_axis_name="s",
                                   num_cores=1)


def _sc_offload(f):
    return compute_on.compute_on2(
        f, compute_type="tpu_sparsecore",
        out_memory_spaces=_jcore.MemorySpace.Device)


@functools.partial(
    pl.kernel,
    out_type=jax.ShapeDtypeStruct((_NC, _N, 16), jnp.float32),
    mesh=_sc_mesh,
    scratch_types=[
        pltpu.VMEM((_NCHUNK, _CH), jnp.int32),
        pltpu.VMEM((_CH, 16), jnp.float32),
        pltpu.VMEM((_RCH, 16), jnp.float32),
        pltpu.VMEM_SHARED((_N, 16), jnp.float32),
    ],
)
def _deg_kernel(dst_hbm, out_hbm, didx, ones_v, zbuf, acc):
    cid = lax.axis_index("c")
    sid = lax.axis_index("s")
    w = cid * _NS + sid

    @pl.loop(0, _RCH)
    def _zero(i):
        zbuf[i] = jnp.zeros((16,), jnp.float32)

    @pl.loop(0, _CH)
    def _one(i):
        ones_v[i] = jnp.full((16,), 1.0, jnp.float32)

    @pl.loop(sid, _NRCH, step=_NS)
    def _init(r):
        pltpu.sync_copy(zbuf, acc.at[pl.ds(r * _RCH, _RCH)])

    plsc.subcore_barrier()

    pltpu.sync_copy(dst_hbm.at[w], didx)

    @pl.loop(0, _NCHUNK)
    def _scat(j):
        pltpu.sync_copy(ones_v, acc.at[didx.at[j]], add=True)

    plsc.subcore_barrier()

    @pl.loop(sid, _NRCH, step=_NS)
    def _out(r):
        pltpu.sync_copy(acc.at[pl.ds(r * _RCH, _RCH)],
                        out_hbm.at[cid, pl.ds(r * _RCH, _RCH)])


@functools.partial(
    pl.kernel,
    out_type=jax.ShapeDtypeStruct((_N, _H), jnp.float32),
    mesh=_sc_mesh1,
    scratch_types=[
        pltpu.VMEM((1, _CHG, _CHP), jnp.int32),
        pltpu.VMEM((1, _CHG, _CHP), jnp.int32),
        pltpu.VMEM((_RING, _CHP, _H), jnp.float32),
        pltpu.VMEM_SHARED((_N, _H), jnp.float32),
    ] + [pltpu.SemaphoreType.DMA] * (_RING + 2),
)
def _pass_kernel(src_hbm, dst_hbm, table_hbm, out_hbm,
                 sidx, didx, rows, acc, *sems):
    sid = lax.axis_index("s")

    @pl.loop(0, _RCH)
    def _zero(i):
        for k in range(_H // 16):
            rows[0, i, pl.ds(k * 16, 16)] = jnp.zeros((16,), jnp.float32)

    @pl.loop(sid, _NRCH, step=_NS)
    def _init(r):
        pltpu.sync_copy(rows.at[0], acc.at[pl.ds(r * _RCH, _RCH)])

    plsc.subcore_barrier()

    def start(sl, j, b):
        pltpu.make_async_copy(table_hbm.at[sidx.at[sl, j]], rows.at[b],
                              sems[b]).start()

    def wait(b):
        pltpu.make_async_copy(table_hbm.at[sidx.at[0, 0]], rows.at[b],
                              sems[b]).wait()

    def idx_start(g, sl):
        pltpu.make_async_copy(src_hbm.at[sid, g], sidx.at[sl],
                              sems[_RING + sl]).start()
        pltpu.make_async_copy(dst_hbm.at[sid, g], didx.at[sl],
                              sems[_RING + sl]).start()

    def idx_wait(sl):
        pltpu.make_async_copy(src_hbm.at[sid, 0], sidx.at[sl],
                              sems[_RING + sl]).wait()
        pltpu.make_async_copy(dst_hbm.at[sid, 0], didx.at[sl],
                              sems[_RING + sl]).wait()

    def scat(sl, j, b):
        pltpu.sync_copy(rows.at[b], acc.at[didx.at[sl, j]], add=True)

    R = _RING
    bulk = (_CHG - R) // R
    rem = _CHG - R * bulk - R
    for g in range(_NG):
        sl = 0
        idx_start(g, sl)
        idx_wait(sl)
        for b in range(R):
            start(sl, b, b)

        @pl.loop(0, bulk)
        def _main(p):
            j = R * p
            for b in range(R):
                wait(b)
                scat(sl, j + b, b)
                start(sl, j + b + R, b)

        j0 = R * bulk
        for i in range(rem):
            b = i % R
            wait(b)
            scat(sl, j0 + i, b)
            start(sl, j0 + R + i, b)
        for i in range(rem, rem + R):
            b = i % R
            wait(b)
            scat(sl, j0 + i, b)

    plsc.subcore_barrier()

    @pl.loop(sid, _NRCH, step=_NS)
    def _out(r):
        pltpu.sync_copy(acc.at[pl.ds(r * _RCH, _RCH)],
                        out_hbm.at[pl.ds(r * _RCH, _RCH)])


_NT = (((1,), (1,)), ((), ()))


def _prep_body(deg_ref, x_ref, xn_ref, nb_ref):
    deg = deg_ref[0] + deg_ref[1]
    norm = lax.rsqrt(jnp.maximum(deg[:, :1], 1.0))
    nb = jnp.broadcast_to(norm, (_R, _H))
    nb_ref[...] = nb
    xn_ref[...] = x_ref[...] * nb


_prep_call = pl.pallas_call(
    _prep_body,
    grid=(_NB,),
    in_specs=[
        pl.BlockSpec((_NC, _R, 16), lambda i: (0, i, 0)),
        pl.BlockSpec((_R, _D), lambda i: (i, 0)),
    ],
    out_specs=[
        pl.BlockSpec((_R, _D), lambda i: (i, 0)),
        pl.BlockSpec((_R, _H), lambda i: (i, 0)),
    ],
    out_shape=[
        jax.ShapeDtypeStruct((_N, _D), jnp.float32),
        jax.ShapeDtypeStruct((_N, _H), jnp.float32),
    ],
)


def _combine_body(y_ref, nb_ref, z_ref):
    z_ref[...] = y_ref[...] * nb_ref[...]


_combine_call = pl.pallas_call(
    _combine_body,
    grid=(_NB,),
    in_specs=[
        pl.BlockSpec((_R, _H), lambda i: (i, 0)),
        pl.BlockSpec((_R, _H), lambda i: (i, 0)),
    ],
    out_specs=pl.BlockSpec((_R, _H), lambda i: (i, 0)),
    out_shape=jax.ShapeDtypeStruct((_N, _H), jnp.float32),
)


def _b1_body(z_ref, p_ref, w_ref, t_ref, ssum_ref, ssq_ref):
    i = pl.program_id(1)
    z = z_ref[...]
    zp = lax.dot_general(z, p_ref[0], _NT, preferred_element_type=jnp.float32)
    t = lax.dot_general(zp, w_ref[0], _NT, preferred_element_type=jnp.float32)
    t_ref[0] = t

    @pl.when(i == 0)
    def _():
        ssum_ref[0] = jnp.zeros((1, _H), jnp.float32)
        ssq_ref[0] = jnp.zeros((1, _H), jnp.float32)

    ssum_ref[0] += jnp.sum(t, axis=0, keepdims=True)
    ssq_ref[0] += jnp.sum(t * t, axis=0, keepdims=True)


_b1_call = pl.pallas_call(
    _b1_body,
    grid=(_S, _NB),
    in_specs=[
        pl.BlockSpec((_R, _H), lambda s, i: (i, 0)),
        pl.BlockSpec((1, _H, _D), lambda s, i: (s, 0, 0)),
        pl.BlockSpec((1, _H, _H), lambda s, i: (s, 0, 0)),
    ],
    out_specs=[
        pl.BlockSpec((1, _R, _H), lambda s, i: (s, i, 0)),
        pl.BlockSpec((1, 1, _H), lambda s, i: (s, 0, 0)),
        pl.BlockSpec((1, 1, _H), lambda s, i: (s, 0, 0)),
    ],
    out_shape=[
        jax.ShapeDtypeStruct((_S, _N, _H), jnp.float32),
        jax.ShapeDtypeStruct((_S, 1, _H), jnp.float32),
        jax.ShapeDtypeStruct((_S, 1, _H), jnp.float32),
    ],
)


def _bn_coeffs(ssum, ssq, gamma, beta):
    mean = ssum * (1.0 / _N)
    var = ssq * (1.0 / _N) - mean * mean
    a = gamma * lax.rsqrt(var + 1e-5)
    b = beta - mean * a
    return a, b


def _b2_body(t_ref, ssum_ref, ssq_ref, g_ref, b_ref, nb_ref, hn_ref):
    a, b = _bn_coeffs(ssum_ref[0], ssq_ref[0], g_ref[0], b_ref[0])
    hn_ref[0] = jnp.maximum(t_ref[0] * a + b, 0.0) * nb_ref[...]


_b2_call = pl.pallas_call(
    _b2_body,
    grid=(_S, _NB),
    in_specs=[
        pl.BlockSpec((1, _R, _H), lambda s, i: (s, i, 0)),
        pl.BlockSpec((1, 1, _H), lambda s, i: (s, 0, 0)),
        pl.BlockSpec((1, 1, _H), lambda s, i: (s, 0, 0)),
        pl.BlockSpec((1, 1, _H), lambda s, i: (s, 0, 0)),
        pl.BlockSpec((1, 1, _H), lambda s, i: (s, 0, 0)),
        pl.BlockSpec((_R, _H), lambda s, i: (i, 0)),
    ],
    out_specs=pl.BlockSpec((1, _R, _H), lambda s, i: (s, i, 0)),
    out_shape=jax.ShapeDtypeStruct((_S, _N, _H), jnp.float32),
)


def _c1_body(z1_ref, w_ref, u_ref, csum_ref, csq_ref):
    i = pl.program_id(1)
    u = lax.dot_general(z1_ref[0], w_ref[0], _NT,
                        preferred_element_type=jnp.float32)
    u_ref[0] = u

    @pl.when(i == 0)
    def _():
        csum_ref[0] = jnp.zeros((1, _H), jnp.float32)
        csq_ref[0] = jnp.zeros((1, _H), jnp.float32)

    csum_ref[0] += jnp.sum(u, axis=0, keepdims=True)
    csq_ref[0] += jnp.sum(u * u, axis=0, keepdims=True)


_c1_call = pl.pallas_call(
    _c1_body,
    grid=(_S, _NB),
    in_specs=[
        pl.BlockSpec((1, _R, _H), lambda s, i: (s, i, 0)),
        pl.BlockSpec((1, _H, _H), lambda s, i: (s, 0, 0)),
    ],
    out_specs=[
        pl.BlockSpec((1, _R, _H), lambda s, i: (s, i, 0)),
        pl.BlockSpec((1, 1, _H), lambda s, i: (s, 0, 0)),
        pl.BlockSpec((1, 1, _H), lambda s, i: (s, 0, 0)),
    ],
    out_shape=[
        jax.ShapeDtypeStruct((_S, _N, _H), jnp.float32),
        jax.ShapeDtypeStruct((_S, 1, _H), jnp.float32),
        jax.ShapeDtypeStruct((_S, 1, _H), jnp.float32),
    ],
)


def _pool_body(u_ref, csum_ref, csq_ref, g_ref, b_ref, batch_ref,
               pooled_ref, cnt_ref):
    i = pl.program_id(0)
    havg = jnp.zeros((_R, _H), jnp.float32)
    for s in range(_S):
        a, b = _bn_coeffs(csum_ref[s], csq_ref[s], g_ref[s], b_ref[s])
        havg = havg + jnp.maximum(u_ref[s] * a + b, 0.0)
    havg = havg * (1.0 / _S)

    bvals = batch_ref[0]
    rows = lax.broadcasted_iota(jnp.int32, (_G, _R), 0)
    oh = (bvals == rows).astype(jnp.float32)

    @pl.when(i == 0)
    def _():
        pooled_ref[...] = jnp.zeros((_G, _H), jnp.float32)
        cnt_ref[...] = jnp.zeros((_G, 1), jnp.float32)

    pooled_ref[...] += jnp.dot(oh, havg, preferred_element_type=jnp.float32)
    cnt_ref[...] += jnp.sum(oh, axis=1, keepdims=True)


_pool_call = pl.pallas_call(
    _pool_body,
    grid=(_NB,),
    in_specs=[
        pl.BlockSpec((_S, _R, _H), lambda i: (0, i, 0)),
        pl.BlockSpec((_S, 1, _H), lambda i: (0, 0, 0)),
        pl.BlockSpec((_S, 1, _H), lambda i: (0, 0, 0)),
        pl.BlockSpec((_S, 1, _H), lambda i: (0, 0, 0)),
        pl.BlockSpec((_S, 1, _H), lambda i: (0, 0, 0)),
        pl.BlockSpec((1, 1, _R), lambda i: (i, 0, 0)),
    ],
    out_specs=[
        pl.BlockSpec((_G, _H), lambda i: (0, 0)),
        pl.BlockSpec((_G, 1), lambda i: (0, 0)),
    ],
    out_shape=[
        jax.ShapeDtypeStruct((_G, _H), jnp.float32),
        jax.ShapeDtypeStruct((_G, 1), jnp.float32),
    ],
)


def _head_body(pooled_ref, cnt_ref, fc1w_ref, fc1b_ref, outw_ref, outb_ref,
               out_ref):
    y = pooled_ref[...] / jnp.maximum(cnt_ref[...], 1.0)
    nrm = jnp.sqrt(jnp.sum(y * y, axis=1, keepdims=True))
    h = y / jnp.maximum(nrm, 1e-12)
    h1 = lax.dot_general(h, fc1w_ref[...], _NT,
                         preferred_element_type=jnp.float32) + fc1b_ref[...]
    h1 = jnp.where(h1 >= 0, h1, 0.01 * h1)
    o = jnp.sum(h1 * outw_ref[...], axis=1, keepdims=True) + outb_ref[...]
    out_ref[...] = 1.0 / (1.0 + jnp.exp(-o))


_head_call = pl.pallas_call(
    _head_body,
    out_shape=jax.ShapeDtypeStruct((_G, 1), jnp.float32),
)


def kernel(x, fc_proj_W, gcn_W, bn_gamma, bn_beta, fc1_W, fc1_b, out_W,
           out_b, edge_index, batch):
    src = edge_index[0].reshape(_NS, _NG, _CHG, _CHP)
    dst = edge_index[1].reshape(_NS, _NG, _CHG, _CHP)

    deg2 = _sc_offload(_deg_kernel)(edge_index[1].reshape(_NW, _NCHUNK, _CH))
    xn, normb = _prep_call(deg2, x)

    y2 = _sc_offload(_pass_kernel)(src, dst, xn)
    z = _combine_call(y2, normb)

    t, ssum, ssq = _b1_call(z, fc_proj_W, gcn_W[:, 0])
    hn = _b2_call(t, ssum, ssq,
                  bn_gamma[:, 0].reshape(_S, 1, _H),
                  bn_beta[:, 0].reshape(_S, 1, _H), normb)

    z1 = jnp.stack(
        [_combine_call(_sc_offload(_pass_kernel)(src, dst, hn[s]), normb)
         for s in range(_S)])

    u, csum, csq = _c1_call(z1, gcn_W[:, 1])
    pooled, cnt = _pool_call(u, csum, csq,
                             bn_gamma[:, 1].reshape(_S, 1, _H),
                             bn_beta[:, 1].reshape(_S, 1, _H),
                             batch.reshape(_NB, 1, _R))
    return _head_call(pooled, cnt, fc1_W, fc1_b.reshape(1, _H), out_W,
                      out_b.reshape(1, 1))


# speedup vs baseline: 1.0420x; 1.0035x over previous
import functools

import jax
import jax.numpy as jnp
from jax import lax
from jax._src import core as _jcore
from jax.experimental import compute_on
from jax.experimental import pallas as pl
from jax.experimental.pallas import tpu as pltpu
from jax.experimental.pallas import tpu_sc as plsc

_N = 10000
_E = 320000
_D = 128
_H = 128
_S = 5
_G = 256

_NC = 2
_NS = 16
_NW = _NC * _NS
_EPW = _E // _NW
_CH = 40
_NCHUNK = _EPW // _CH
_CHP = 80
_EPS = _E // _NS
_NCHP = _EPS // _CHP
_NG = 5
_CHG = _NCHP // _NG
_RING = 3
_RCH = 80
_NRCH = _N // _RCH

_R = 1000
_NB = _N // _R

_sc_mesh = plsc.VectorSubcoreMesh(core_axis_name="c", subcore_axis_name="s")
_sc_mesh1 = plsc.VectorSubcoreMesh(core_axis_name="c", subcore_axis_name="s",
                                   num_cores=1)


def _sc_offload(f):
    return compute_on.compute_on2(
        f, compute_type="tpu_sparsecore",
        out_memory_spaces=_jcore.MemorySpace.Device)


@functools.partial(
    pl.kernel,
    out_type=jax.ShapeDtypeStruct((_NC, _N, 16), jnp.float32),
    mesh=_sc_mesh,
    scratch_types=[
        pltpu.VMEM((_NCHUNK, _CH), jnp.int32),
        pltpu.VMEM((_CH, 16), jnp.float32),
        pltpu.VMEM((_RCH, 16), jnp.float32),
        pltpu.VMEM_SHARED((_N, 16), jnp.float32),
    ],
)
def _deg_kernel(dst_hbm, out_hbm, didx, ones_v, zbuf, acc):
    cid = lax.axis_index("c")
    sid = lax.axis_index("s")
    w = cid * _NS + sid

    @pl.loop(0, _RCH)
    def _zero(i):
        zbuf[i] = jnp.zeros((16,), jnp.float32)

    @pl.loop(0, _CH)
    def _one(i):
        ones_v[i] = jnp.full((16,), 1.0, jnp.float32)

    @pl.loop(sid, _NRCH, step=_NS)
    def _init(r):
        pltpu.sync_copy(zbuf, acc.at[pl.ds(r * _RCH, _RCH)])

    plsc.subcore_barrier()

    pltpu.sync_copy(dst_hbm.at[w], didx)

    @pl.loop(0, _NCHUNK)
    def _scat(j):
        pltpu.sync_copy(ones_v, acc.at[didx.at[j]], add=True)

    plsc.subcore_barrier()

    @pl.loop(sid, _NRCH, step=_NS)
    def _out(r):
        pltpu.sync_copy(acc.at[pl.ds(r * _RCH, _RCH)],
                        out_hbm.at[cid, pl.ds(r * _RCH, _RCH)])


@functools.partial(
    pl.kernel,
    out_type=jax.ShapeDtypeStruct((_N, _H), jnp.float32),
    mesh=_sc_mesh1,
    scratch_types=[
        pltpu.VMEM((1, _CHG, _CHP), jnp.int32),
        pltpu.VMEM((1, _CHG, _CHP), jnp.int32),
        pltpu.VMEM((_RING, _CHP, _H), jnp.float32),
        pltpu.VMEM_SHARED((_N, _H), jnp.float32),
    ] + [pltpu.SemaphoreType.DMA] * (_RING + 2),
)
def _pass_kernel(src_hbm, dst_hbm, table_hbm, out_hbm,
                 sidx, didx, rows, acc, *sems):
    sid = lax.axis_index("s")

    @pl.loop(0, _RCH)
    def _zero(i):
        for k in range(_H // 16):
            rows[0, i, pl.ds(k * 16, 16)] = jnp.zeros((16,), jnp.float32)

    @pl.loop(sid, _NRCH, step=_NS)
    def _init(r):
        pltpu.sync_copy(rows.at[0], acc.at[pl.ds(r * _RCH, _RCH)])

    plsc.subcore_barrier()

    def start(sl, j, b):
        pltpu.make_async_copy(table_hbm.at[sidx.at[sl, j]], rows.at[b],
                              sems[b]).start()

    def wait(b):
        pltpu.make_async_copy(table_hbm.at[sidx.at[0, 0]], rows.at[b],
                              sems[b]).wait()

    def idx_start(g, sl):
        pltpu.make_async_copy(src_hbm.at[sid, g], sidx.at[sl],
                              sems[_RING + sl]).start()
        pltpu.make_async_copy(dst_hbm.at[sid, g], didx.at[sl],
                              sems[_RING + sl]).start()

    def idx_wait(sl):
        pltpu.make_async_copy(src_hbm.at[sid, 0], sidx.at[sl],
                              sems[_RING + sl]).wait()
        pltpu.make_async_copy(dst_hbm.at[sid, 0], didx.at[sl],
                              sems[_RING + sl]).wait()

    def scat(sl, j, b):
        pltpu.sync_copy(rows.at[b], acc.at[didx.at[sl, j]], add=True)

    R = _RING
    bulk = (_CHG - R) // R
    rem = _CHG - R * bulk - R
    for g in range(_NG):
        sl = 0
        idx_start(g, sl)
        idx_wait(sl)
        for b in range(R):
            start(sl, b, b)

        @pl.loop(0, bulk)
        def _main(p):
            j = R * p
            for b in range(R):
                wait(b)
                scat(sl, j + b, b)
                start(sl, j + b + R, b)

        j0 = R * bulk
        for i in range(rem):
            b = i % R
            wait(b)
            scat(sl, j0 + i, b)
            start(sl, j0 + R + i, b)
        for i in range(rem, rem + R):
            b = i % R
            wait(b)
            scat(sl, j0 + i, b)

    plsc.subcore_barrier()

    @pl.loop(sid, _NRCH, step=_NS)
    def _out(r):
        pltpu.sync_copy(acc.at[pl.ds(r * _RCH, _RCH)],
                        out_hbm.at[pl.ds(r * _RCH, _RCH)])


@functools.partial(
    pl.kernel,
    out_type=jax.ShapeDtypeStruct((_S, _N, _H), jnp.float32),
    mesh=_sc_mesh1,
    scratch_types=[
        pltpu.VMEM((1, _CHG, _CHP), jnp.int32),
        pltpu.VMEM((1, _CHG, _CHP), jnp.int32),
        pltpu.VMEM((_RING, _CHP, _H), jnp.float32),
        pltpu.VMEM_SHARED((_N, _H), jnp.float32),
    ] + [pltpu.SemaphoreType.DMA] * (_RING + 2),
)
def _pass5_kernel(src_hbm, dst_hbm, tables_hbm, out_hbm,
                  sidx, didx, rows, acc, *sems):
    sid = lax.axis_index("s")

    @pl.loop(0, _S)
    def _scale(sc):
        table_hbm = tables_hbm.at[sc]

        @pl.loop(0, _RCH)
        def _zero(i):
            for k in range(_H // 16):
                rows[0, i, pl.ds(k * 16, 16)] = jnp.zeros((16,), jnp.float32)

        @pl.loop(sid, _NRCH, step=_NS)
        def _init(r):
            pltpu.sync_copy(rows.at[0], acc.at[pl.ds(r * _RCH, _RCH)])

        plsc.subcore_barrier()

        def start(sl, j, b):
            pltpu.make_async_copy(table_hbm.at[sidx.at[sl, j]], rows.at[b],
                                  sems[b]).start()

        def wait(b):
            pltpu.make_async_copy(table_hbm.at[sidx.at[0, 0]], rows.at[b],
                                  sems[b]).wait()

        def idx_start(g, sl):
            pltpu.make_async_copy(src_hbm.at[sid, g], sidx.at[sl],
                                  sems[_RING + sl]).start()
            pltpu.make_async_copy(dst_hbm.at[sid, g], didx.at[sl],
                                  sems[_RING + sl]).start()

        def idx_wait(sl):
            pltpu.make_async_copy(src_hbm.at[sid, 0], sidx.at[sl],
                                  sems[_RING + sl]).wait()
            pltpu.make_async_copy(dst_hbm.at[sid, 0], didx.at[sl],
                                  sems[_RING + sl]).wait()

        def scat(sl, j, b):
            pltpu.sync_copy(rows.at[b], acc.at[didx.at[sl, j]], add=True)

        R = _RING
        bulk = (_CHG - R) // R
        rem = _CHG - R * bulk - R
        for g in range(_NG):
            sl = 0
            idx_start(g, sl)
            idx_wait(sl)
            for b in range(R):
                start(sl, b, b)

            @pl.loop(0, bulk)
            def _main(p):
                j = R * p
                for b in range(R):
                    wait(b)
                    scat(sl, j + b, b)
                    start(sl, j + b + R, b)

            j0 = R * bulk
            for i in range(rem):
                b = i % R
                wait(b)
                scat(sl, j0 + i, b)
                start(sl, j0 + R + i, b)
            for i in range(rem, rem + R):
                b = i % R
                wait(b)
                scat(sl, j0 + i, b)

        plsc.subcore_barrier()

        @pl.loop(sid, _NRCH, step=_NS)
        def _out(r):
            pltpu.sync_copy(acc.at[pl.ds(r * _RCH, _RCH)],
                            out_hbm.at[sc, pl.ds(r * _RCH, _RCH)])

        plsc.subcore_barrier()


_NT = (((1,), (1,)), ((), ()))


def _prep_body(deg_ref, x_ref, xn_ref, nb_ref):
    deg = deg_ref[0] + deg_ref[1]
    norm = lax.rsqrt(jnp.maximum(deg[:, :1], 1.0))
    nb = jnp.broadcast_to(norm, (_R, _H))
    nb_ref[...] = nb
    xn_ref[...] = x_ref[...] * nb


_prep_call = pl.pallas_call(
    _prep_body,
    grid=(_NB,),
    in_specs=[
        pl.BlockSpec((_NC, _R, 16), lambda i: (0, i, 0)),
        pl.BlockSpec((_R, _D), lambda i: (i, 0)),
    ],
    out_specs=[
        pl.BlockSpec((_R, _D), lambda i: (i, 0)),
        pl.BlockSpec((_R, _H), lambda i: (i, 0)),
    ],
    out_shape=[
        jax.ShapeDtypeStruct((_N, _D), jnp.float32),
        jax.ShapeDtypeStruct((_N, _H), jnp.float32),
    ],
)


def _combine_body(y_ref, nb_ref, z_ref):
    z_ref[...] = y_ref[...] * nb_ref[...]


_combine_call = pl.pallas_call(
    _combine_body,
    grid=(_NB,),
    in_specs=[
        pl.BlockSpec((_R, _H), lambda i: (i, 0)),
        pl.BlockSpec((_R, _H), lambda i: (i, 0)),
    ],
    out_specs=pl.BlockSpec((_R, _H), lambda i: (i, 0)),
    out_shape=jax.ShapeDtypeStruct((_N, _H), jnp.float32),
)


def _combine5_body(y_ref, nb_ref, z_ref):
    z_ref[0] = y_ref[0] * nb_ref[...]


_combine5_call = pl.pallas_call(
    _combine5_body,
    grid=(_S, _NB),
    in_specs=[
        pl.BlockSpec((1, _R, _H), lambda s, i: (s, i, 0)),
        pl.BlockSpec((_R, _H), lambda s, i: (i, 0)),
    ],
    out_specs=pl.BlockSpec((1, _R, _H), lambda s, i: (s, i, 0)),
    out_shape=jax.ShapeDtypeStruct((_S, _N, _H), jnp.float32),
)


def _b1_body(z_ref, p_ref, w_ref, t_ref, ssum_ref, ssq_ref):
    i = pl.program_id(1)
    z = z_ref[...]
    zp = lax.dot_general(z, p_ref[0], _NT, preferred_element_type=jnp.float32)
    t = lax.dot_general(zp, w_ref[0], _NT, preferred_element_type=jnp.float32)
    t_ref[0] = t

    @pl.when(i == 0)
    def _():
        ssum_ref[0] = jnp.zeros((1, _H), jnp.float32)
        ssq_ref[0] = jnp.zeros((1, _H), jnp.float32)

    ssum_ref[0] += jnp.sum(t, axis=0, keepdims=True)
    ssq_ref[0] += jnp.sum(t * t, axis=0, keepdims=True)


_b1_call = pl.pallas_call(
    _b1_body,
    grid=(_S, _NB),
    in_specs=[
        pl.BlockSpec((_R, _H), lambda s, i: (i, 0)),
        pl.BlockSpec((1, _H, _D), lambda s, i: (s, 0, 0)),
        pl.BlockSpec((1, _H, _H), lambda s, i: (s, 0, 0)),
    ],
    out_specs=[
        pl.BlockSpec((1, _R, _H), lambda s, i: (s, i, 0)),
        pl.BlockSpec((1, 1, _H), lambda s, i: (s, 0, 0)),
        pl.BlockSpec((1, 1, _H), lambda s, i: (s, 0, 0)),
    ],
    out_shape=[
        jax.ShapeDtypeStruct((_S, _N, _H), jnp.float32),
        jax.ShapeDtypeStruct((_S, 1, _H), jnp.float32),
        jax.ShapeDtypeStruct((_S, 1, _H), jnp.float32),
    ],
)


def _bn_coeffs(ssum, ssq, gamma, beta):
    mean = ssum * (1.0 / _N)
    var = ssq * (1.0 / _N) - mean * mean
    a = gamma * lax.rsqrt(var + 1e-5)
    b = beta - mean * a
    return a, b


def _b2_body(t_ref, ssum_ref, ssq_ref, g_ref, b_ref, nb_ref, hn_ref):
    a, b = _bn_coeffs(ssum_ref[0], ssq_ref[0], g_ref[0], b_ref[0])
    hn_ref[0] = jnp.maximum(t_ref[0] * a + b, 0.0) * nb_ref[...]


_b2_call = pl.pallas_call(
    _b2_body,
    grid=(_S, _NB),
    in_specs=[
        pl.BlockSpec((1, _R, _H), lambda s, i: (s, i, 0)),
        pl.BlockSpec((1, 1, _H), lambda s, i: (s, 0, 0)),
        pl.BlockSpec((1, 1, _H), lambda s, i: (s, 0, 0)),
        pl.BlockSpec((1, 1, _H), lambda s, i: (s, 0, 0)),
        pl.BlockSpec((1, 1, _H), lambda s, i: (s, 0, 0)),
        pl.BlockSpec((_R, _H), lambda s, i: (i, 0)),
    ],
    out_specs=pl.BlockSpec((1, _R, _H), lambda s, i: (s, i, 0)),
    out_shape=jax.ShapeDtypeStruct((_S, _N, _H), jnp.float32),
)


def _c1_body(z1_ref, w_ref, u_ref, csum_ref, csq_ref):
    i = pl.program_id(1)
    u = lax.dot_general(z1_ref[0], w_ref[0], _NT,
                        preferred_element_type=jnp.float32)
    u_ref[0] = u

    @pl.when(i == 0)
    def _():
        csum_ref[0] = jnp.zeros((1, _H), jnp.float32)
        csq_ref[0] = jnp.zeros((1, _H), jnp.float32)

    csum_ref[0] += jnp.sum(u, axis=0, keepdims=True)
    csq_ref[0] += jnp.sum(u * u, axis=0, keepdims=True)


_c1_call = pl.pallas_call(
    _c1_body,
    grid=(_S, _NB),
    in_specs=[
        pl.BlockSpec((1, _R, _H), lambda s, i: (s, i, 0)),
        pl.BlockSpec((1, _H, _H), lambda s, i: (s, 0, 0)),
    ],
    out_specs=[
        pl.BlockSpec((1, _R, _H), lambda s, i: (s, i, 0)),
        pl.BlockSpec((1, 1, _H), lambda s, i: (s, 0, 0)),
        pl.BlockSpec((1, 1, _H), lambda s, i: (s, 0, 0)),
    ],
    out_shape=[
        jax.ShapeDtypeStruct((_S, _N, _H), jnp.float32),
        jax.ShapeDtypeStruct((_S, 1, _H), jnp.float32),
        jax.ShapeDtypeStruct((_S, 1, _H), jnp.float32),
    ],
)


def _pool_body(u_ref, csum_ref, csq_ref, g_ref, b_ref, batch_ref,
               pooled_ref, cnt_ref):
    i = pl.program_id(0)
    havg = jnp.zeros((_R, _H), jnp.float32)
    for s in range(_S):
        a, b = _bn_coeffs(csum_ref[s], csq_ref[s], g_ref[s], b_ref[s])
        havg = havg + jnp.maximum(u_ref[s] * a + b, 0.0)
    havg = havg * (1.0 / _S)

    bvals = batch_ref[0]
    rows = lax.broadcasted_iota(jnp.int32, (_G, _R), 0)
    oh = (bvals == rows).astype(jnp.float32)

    @pl.when(i == 0)
    def _():
        pooled_ref[...] = jnp.zeros((_G, _H), jnp.float32)
        cnt_ref[...] = jnp.zeros((_G, 1), jnp.float32)

    pooled_ref[...] += jnp.dot(oh, havg, preferred_element_type=jnp.float32)
    cnt_ref[...] += jnp.sum(oh, axis=1, keepdims=True)


_pool_call = pl.pallas_call(
    _pool_body,
    grid=(_NB,),
    in_specs=[
        pl.BlockSpec((_S, _R, _H), lambda i: (0, i, 0)),
        pl.BlockSpec((_S, 1, _H), lambda i: (0, 0, 0)),
        pl.BlockSpec((_S, 1, _H), lambda i: (0, 0, 0)),
        pl.BlockSpec((_S, 1, _H), lambda i: (0, 0, 0)),
        pl.BlockSpec((_S, 1, _H), lambda i: (0, 0, 0)),
        pl.BlockSpec((1, 1, _R), lambda i: (i, 0, 0)),
    ],
    out_specs=[
        pl.BlockSpec((_G, _H), lambda i: (0, 0)),
        pl.BlockSpec((_G, 1), lambda i: (0, 0)),
    ],
    out_shape=[
        jax.ShapeDtypeStruct((_G, _H), jnp.float32),
        jax.ShapeDtypeStruct((_G, 1), jnp.float32),
    ],
)


def _head_body(pooled_ref, cnt_ref, fc1w_ref, fc1b_ref, outw_ref, outb_ref,
               out_ref):
    y = pooled_ref[...] / jnp.maximum(cnt_ref[...], 1.0)
    nrm = jnp.sqrt(jnp.sum(y * y, axis=1, keepdims=True))
    h = y / jnp.maximum(nrm, 1e-12)
    h1 = lax.dot_general(h, fc1w_ref[...], _NT,
                         preferred_element_type=jnp.float32) + fc1b_ref[...]
    h1 = jnp.where(h1 >= 0, h1, 0.01 * h1)
    o = jnp.sum(h1 * outw_ref[...], axis=1, keepdims=True) + outb_ref[...]
    out_ref[...] = 1.0 / (1.0 + jnp.exp(-o))


_head_call = pl.pallas_call(
    _head_body,
    out_shape=jax.ShapeDtypeStruct((_G, 1), jnp.float32),
)


def kernel(x, fc_proj_W, gcn_W, bn_gamma, bn_beta, fc1_W, fc1_b, out_W,
           out_b, edge_index, batch):
    src = edge_index[0].reshape(_NS, _NG, _CHG, _CHP)
    dst = edge_index[1].reshape(_NS, _NG, _CHG, _CHP)

    deg2 = _sc_offload(_deg_kernel)(edge_index[1].reshape(_NW, _NCHUNK, _CH))
    xn, normb = _prep_call(deg2, x)

    y2 = _sc_offload(_pass_kernel)(src, dst, xn)
    z = _combine_call(y2, normb)

    t, ssum, ssq = _b1_call(z, fc_proj_W, gcn_W[:, 0])
    hn = _b2_call(t, ssum, ssq,
                  bn_gamma[:, 0].reshape(_S, 1, _H),
                  bn_beta[:, 0].reshape(_S, 1, _H), normb)

    agg5 = _sc_offload(_pass5_kernel)(src, dst, hn)
    z1 = _combine5_call(agg5, normb)

    u, csum, csq = _c1_call(z1, gcn_W[:, 1])
    pooled, cnt = _pool_call(u, csum, csq,
                             bn_gamma[:, 1].reshape(_S, 1, _H),
                             bn_beta[:, 1].reshape(_S, 1, _H),
                             batch.reshape(_NB, 1, _R))
    return _head_call(pooled, cnt, fc1_W, fc1_b.reshape(1, _H), out_W,
                      out_b.reshape(1, 1))


# speedup vs baseline: 1.0702x; 1.0270x over previous
import functools

import jax
import jax.numpy as jnp
from jax import lax
from jax._src import core as _jcore
from jax.experimental import compute_on
from jax.experimental import pallas as pl
from jax.experimental.pallas import tpu as pltpu
from jax.experimental.pallas import tpu_sc as plsc

_N = 10000
_E = 320000
_D = 128
_H = 128
_S = 5
_G = 256

_NC = 2
_NS = 16
_NW = _NC * _NS
_EPW = _E // _NW
_CH = 40
_NCHUNK = _EPW // _CH
_CHP = 80
_EPS = _E // _NS
_NCHP = _EPS // _CHP
_NG = 5
_CHG = _NCHP // _NG
_RING = 3
_RCH = 80
_NRCH = _N // _RCH

_R = 1000
_NB = _N // _R

_sc_mesh = plsc.VectorSubcoreMesh(core_axis_name="c", subcore_axis_name="s")
_sc_mesh1 = plsc.VectorSubcoreMesh(core_axis_name="c", subcore_axis_name="s",
                                   num_cores=1)


def _sc_offload(f):
    return compute_on.compute_on2(
        f, compute_type="tpu_sparsecore",
        out_memory_spaces=_jcore.MemorySpace.Device)


@functools.partial(
    pl.kernel,
    out_type=jax.ShapeDtypeStruct((_NC, _N, 16), jnp.float32),
    mesh=_sc_mesh,
    scratch_types=[
        pltpu.VMEM((_NCHUNK, _CH), jnp.int32),
        pltpu.VMEM((_CH, 16), jnp.float32),
        pltpu.VMEM((_RCH, 16), jnp.float32),
        pltpu.VMEM_SHARED((_N, 16), jnp.float32),
    ],
)
def _deg_kernel(dst_hbm, out_hbm, didx, ones_v, zbuf, acc):
    cid = lax.axis_index("c")
    sid = lax.axis_index("s")
    w = cid * _NS + sid

    @pl.loop(0, _RCH)
    def _zero(i):
        zbuf[i] = jnp.zeros((16,), jnp.float32)

    @pl.loop(0, _CH)
    def _one(i):
        ones_v[i] = jnp.full((16,), 1.0, jnp.float32)

    @pl.loop(sid, _NRCH, step=_NS)
    def _init(r):
        pltpu.sync_copy(zbuf, acc.at[pl.ds(r * _RCH, _RCH)])

    plsc.subcore_barrier()

    pltpu.sync_copy(dst_hbm.at[w], didx)

    @pl.loop(0, _NCHUNK)
    def _scat(j):
        pltpu.sync_copy(ones_v, acc.at[didx.at[j]], add=True)

    plsc.subcore_barrier()

    @pl.loop(sid, _NRCH, step=_NS)
    def _out(r):
        pltpu.sync_copy(acc.at[pl.ds(r * _RCH, _RCH)],
                        out_hbm.at[cid, pl.ds(r * _RCH, _RCH)])


@functools.partial(
    pl.kernel,
    out_type=jax.ShapeDtypeStruct((_N, _H), jnp.float32),
    mesh=_sc_mesh1,
    scratch_types=[
        pltpu.VMEM((1, _CHG, _CHP), jnp.int32),
        pltpu.VMEM((1, _CHG, _CHP), jnp.int32),
        pltpu.VMEM((_RING, _CHP, _H), jnp.float32),
        pltpu.VMEM_SHARED((_N, _H), jnp.float32),
    ] + [pltpu.SemaphoreType.DMA] * (_RING + 2),
)
def _pass_kernel(src_hbm, dst_hbm, table_hbm, out_hbm,
                 sidx, didx, rows, acc, *sems):
    sid = lax.axis_index("s")

    @pl.loop(0, _RCH)
    def _zero(i):
        for k in range(_H // 16):
            rows[0, i, pl.ds(k * 16, 16)] = jnp.zeros((16,), jnp.float32)

    @pl.loop(sid, _NRCH, step=_NS)
    def _init(r):
        pltpu.sync_copy(rows.at[0], acc.at[pl.ds(r * _RCH, _RCH)])

    plsc.subcore_barrier()

    def start(sl, j, b):
        pltpu.make_async_copy(table_hbm.at[sidx.at[sl, j]], rows.at[b],
                              sems[b]).start()

    def wait(b):
        pltpu.make_async_copy(table_hbm.at[sidx.at[0, 0]], rows.at[b],
                              sems[b]).wait()

    def idx_start(g, sl):
        pltpu.make_async_copy(src_hbm.at[sid, g], sidx.at[sl],
                              sems[_RING + sl]).start()
        pltpu.make_async_copy(dst_hbm.at[sid, g], didx.at[sl],
                              sems[_RING + sl]).start()

    def idx_wait(sl):
        pltpu.make_async_copy(src_hbm.at[sid, 0], sidx.at[sl],
                              sems[_RING + sl]).wait()
        pltpu.make_async_copy(dst_hbm.at[sid, 0], didx.at[sl],
                              sems[_RING + sl]).wait()

    def scat(sl, j, b):
        pltpu.sync_copy(rows.at[b], acc.at[didx.at[sl, j]], add=True)

    R = _RING
    bulk = (_CHG - R) // R
    rem = _CHG - R * bulk - R
    for g in range(_NG):
        sl = 0
        idx_start(g, sl)
        idx_wait(sl)
        for b in range(R):
            start(sl, b, b)

        @pl.loop(0, bulk)
        def _main(p):
            j = R * p
            for b in range(R):
                wait(b)
                scat(sl, j + b, b)
                start(sl, j + b + R, b)

        j0 = R * bulk
        for i in range(rem):
            b = i % R
            wait(b)
            scat(sl, j0 + i, b)
            start(sl, j0 + R + i, b)
        for i in range(rem, rem + R):
            b = i % R
            wait(b)
            scat(sl, j0 + i, b)

    plsc.subcore_barrier()

    @pl.loop(sid, _NRCH, step=_NS)
    def _out(r):
        pltpu.sync_copy(acc.at[pl.ds(r * _RCH, _RCH)],
                        out_hbm.at[pl.ds(r * _RCH, _RCH)])


@functools.partial(
    pl.kernel,
    out_type=jax.ShapeDtypeStruct((_S, _N, _H), jnp.float32),
    mesh=_sc_mesh1,
    scratch_types=[
        pltpu.VMEM((1, _CHG, _CHP), jnp.int32),
        pltpu.VMEM((1, _CHG, _CHP), jnp.int32),
        pltpu.VMEM((_RING, _CHP, _H), jnp.float32),
        pltpu.VMEM_SHARED((_N, _H), jnp.float32),
    ] + [pltpu.SemaphoreType.DMA] * (_RING + 2),
)
def _pass5_kernel(src_hbm, dst_hbm, tables_hbm, out_hbm,
                  sidx, didx, rows, acc, *sems):
    sid = lax.axis_index("s")

    @pl.loop(0, _S)
    def _scale(sc):
        table_hbm = tables_hbm.at[sc]

        @pl.loop(0, _RCH)
        def _zero(i):
            for k in range(_H // 16):
                rows[0, i, pl.ds(k * 16, 16)] = jnp.zeros((16,), jnp.float32)

        @pl.loop(sid, _NRCH, step=_NS)
        def _init(r):
            pltpu.sync_copy(rows.at[0], acc.at[pl.ds(r * _RCH, _RCH)])

        plsc.subcore_barrier()

        def start(sl, j, b):
            pltpu.make_async_copy(table_hbm.at[sidx.at[sl, j]], rows.at[b],
                                  sems[b]).start()

        def wait(b):
            pltpu.make_async_copy(table_hbm.at[sidx.at[0, 0]], rows.at[b],
                                  sems[b]).wait()

        def idx_start(g, sl):
            pltpu.make_async_copy(src_hbm.at[sid, g], sidx.at[sl],
                                  sems[_RING + sl]).start()
            pltpu.make_async_copy(dst_hbm.at[sid, g], didx.at[sl],
                                  sems[_RING + sl]).start()

        def idx_wait(sl):
            pltpu.make_async_copy(src_hbm.at[sid, 0], sidx.at[sl],
                                  sems[_RING + sl]).wait()
            pltpu.make_async_copy(dst_hbm.at[sid, 0], didx.at[sl],
                                  sems[_RING + sl]).wait()

        def scat(sl, j, b):
            pltpu.sync_copy(rows.at[b], acc.at[didx.at[sl, j]], add=True)

        R = _RING
        bulk = (_CHG - R) // R
        rem = _CHG - R * bulk - R
        for g in range(_NG):
            sl = 0
            idx_start(g, sl)
            idx_wait(sl)
            for b in range(R):
                start(sl, b, b)

            @pl.loop(0, bulk)
            def _main(p):
                j = R * p
                for b in range(R):
                    wait(b)
                    scat(sl, j + b, b)
                    start(sl, j + b + R, b)

            j0 = R * bulk
            for i in range(rem):
                b = i % R
                wait(b)
                scat(sl, j0 + i, b)
                start(sl, j0 + R + i, b)
            for i in range(rem, rem + R):
                b = i % R
                wait(b)
                scat(sl, j0 + i, b)

        plsc.subcore_barrier()

        @pl.loop(sid, _NRCH, step=_NS)
        def _out(r):
            pltpu.sync_copy(acc.at[pl.ds(r * _RCH, _RCH)],
                            out_hbm.at[sc, pl.ds(r * _RCH, _RCH)])

        plsc.subcore_barrier()


_NT = (((1,), (1,)), ((), ()))


def _prep_body(deg_ref, x_ref, xn_ref, nb_ref):
    deg = deg_ref[0] + deg_ref[1]
    norm = lax.rsqrt(jnp.maximum(deg[:, :1], 1.0))
    nb = jnp.broadcast_to(norm, (_R, _H))
    nb_ref[...] = nb
    xn_ref[...] = x_ref[...] * nb


_prep_call = pl.pallas_call(
    _prep_body,
    grid=(_NB,),
    in_specs=[
        pl.BlockSpec((_NC, _R, 16), lambda i: (0, i, 0)),
        pl.BlockSpec((_R, _D), lambda i: (i, 0)),
    ],
    out_specs=[
        pl.BlockSpec((_R, _D), lambda i: (i, 0)),
        pl.BlockSpec((_R, _H), lambda i: (i, 0)),
    ],
    out_shape=[
        jax.ShapeDtypeStruct((_N, _D), jnp.float32),
        jax.ShapeDtypeStruct((_N, _H), jnp.float32),
    ],
)


def _combine_body(y_ref, nb_ref, z_ref):
    z_ref[...] = y_ref[...] * nb_ref[...]


_combine_call = pl.pallas_call(
    _combine_body,
    grid=(_NB,),
    in_specs=[
        pl.BlockSpec((_R, _H), lambda i: (i, 0)),
        pl.BlockSpec((_R, _H), lambda i: (i, 0)),
    ],
    out_specs=pl.BlockSpec((_R, _H), lambda i: (i, 0)),
    out_shape=jax.ShapeDtypeStruct((_N, _H), jnp.float32),
)


def _combine5_body(y_ref, nb_ref, z_ref):
    z_ref[0] = y_ref[0] * nb_ref[...]


_combine5_call = pl.pallas_call(
    _combine5_body,
    grid=(_S, _NB),
    in_specs=[
        pl.BlockSpec((1, _R, _H), lambda s, i: (s, i, 0)),
        pl.BlockSpec((_R, _H), lambda s, i: (i, 0)),
    ],
    out_specs=pl.BlockSpec((1, _R, _H), lambda s, i: (s, i, 0)),
    out_shape=jax.ShapeDtypeStruct((_S, _N, _H), jnp.float32),
)


def _b1_body(z_ref, p_ref, w_ref, t_ref, ssum_ref, ssq_ref):
    i = pl.program_id(1)
    z = z_ref[...]
    zp = lax.dot_general(z, p_ref[0], _NT, preferred_element_type=jnp.float32)
    t = lax.dot_general(zp, w_ref[0], _NT, preferred_element_type=jnp.float32)
    t_ref[0] = t

    @pl.when(i == 0)
    def _():
        ssum_ref[0] = jnp.zeros((1, _H), jnp.float32)
        ssq_ref[0] = jnp.zeros((1, _H), jnp.float32)

    ssum_ref[0] += jnp.sum(t, axis=0, keepdims=True)
    ssq_ref[0] += jnp.sum(t * t, axis=0, keepdims=True)


_b1_call = pl.pallas_call(
    _b1_body,
    grid=(_S, _NB),
    in_specs=[
        pl.BlockSpec((_R, _H), lambda s, i: (i, 0)),
        pl.BlockSpec((1, _H, _D), lambda s, i: (s, 0, 0)),
        pl.BlockSpec((1, _H, _H), lambda s, i: (s, 0, 0)),
    ],
    out_specs=[
        pl.BlockSpec((1, _R, _H), lambda s, i: (s, i, 0)),
        pl.BlockSpec((1, 1, _H), lambda s, i: (s, 0, 0)),
        pl.BlockSpec((1, 1, _H), lambda s, i: (s, 0, 0)),
    ],
    out_shape=[
        jax.ShapeDtypeStruct((_S, _N, _H), jnp.float32),
        jax.ShapeDtypeStruct((_S, 1, _H), jnp.float32),
        jax.ShapeDtypeStruct((_S, 1, _H), jnp.float32),
    ],
)


def _bn_coeffs(ssum, ssq, gamma, beta):
    mean = ssum * (1.0 / _N)
    var = ssq * (1.0 / _N) - mean * mean
    a = gamma * lax.rsqrt(var + 1e-5)
    b = beta - mean * a
    return a, b


def _b2_body(t_ref, ssum_ref, ssq_ref, g_ref, b_ref, nb_ref, hn_ref):
    a, b = _bn_coeffs(ssum_ref[0], ssq_ref[0], g_ref[0], b_ref[0])
    hn_ref[0] = jnp.maximum(t_ref[0] * a + b, 0.0) * nb_ref[...]


_b2_call = pl.pallas_call(
    _b2_body,
    grid=(_S, _NB),
    in_specs=[
        pl.BlockSpec((1, _R, _H), lambda s, i: (s, i, 0)),
        pl.BlockSpec((1, 1, _H), lambda s, i: (s, 0, 0)),
        pl.BlockSpec((1, 1, _H), lambda s, i: (s, 0, 0)),
        pl.BlockSpec((1, 1, _H), lambda s, i: (s, 0, 0)),
        pl.BlockSpec((1, 1, _H), lambda s, i: (s, 0, 0)),
        pl.BlockSpec((_R, _H), lambda s, i: (i, 0)),
    ],
    out_specs=pl.BlockSpec((1, _R, _H), lambda s, i: (s, i, 0)),
    out_shape=jax.ShapeDtypeStruct((_S, _N, _H), jnp.float32),
)


def _c1_body(agg_ref, nb_ref, w_ref, u_ref, csum_ref, csq_ref):
    i = pl.program_id(1)
    z1 = agg_ref[0] * nb_ref[...]
    u = lax.dot_general(z1, w_ref[0], _NT,
                        preferred_element_type=jnp.float32)
    u_ref[0] = u

    @pl.when(i == 0)
    def _():
        csum_ref[0] = jnp.zeros((1, _H), jnp.float32)
        csq_ref[0] = jnp.zeros((1, _H), jnp.float32)

    csum_ref[0] += jnp.sum(u, axis=0, keepdims=True)
    csq_ref[0] += jnp.sum(u * u, axis=0, keepdims=True)


_c1_call = pl.pallas_call(
    _c1_body,
    grid=(_S, _NB),
    in_specs=[
        pl.BlockSpec((1, _R, _H), lambda s, i: (s, i, 0)),
        pl.BlockSpec((_R, _H), lambda s, i: (i, 0)),
        pl.BlockSpec((1, _H, _H), lambda s, i: (s, 0, 0)),
    ],
    out_specs=[
        pl.BlockSpec((1, _R, _H), lambda s, i: (s, i, 0)),
        pl.BlockSpec((1, 1, _H), lambda s, i: (s, 0, 0)),
        pl.BlockSpec((1, 1, _H), lambda s, i: (s, 0, 0)),
    ],
    out_shape=[
        jax.ShapeDtypeStruct((_S, _N, _H), jnp.float32),
        jax.ShapeDtypeStruct((_S, 1, _H), jnp.float32),
        jax.ShapeDtypeStruct((_S, 1, _H), jnp.float32),
    ],
)


def _pool_body(u_ref, csum_ref, csq_ref, g_ref, b_ref, batch_ref,
               pooled_ref, cnt_ref):
    i = pl.program_id(0)
    havg = jnp.zeros((_R, _H), jnp.float32)
    for s in range(_S):
        a, b = _bn_coeffs(csum_ref[s], csq_ref[s], g_ref[s], b_ref[s])
        havg = havg + jnp.maximum(u_ref[s] * a + b, 0.0)
    havg = havg * (1.0 / _S)

    bvals = batch_ref[0]
    rows = lax.broadcasted_iota(jnp.int32, (_G, _R), 0)
    oh = (bvals == rows).astype(jnp.float32)

    @pl.when(i == 0)
    def _():
        pooled_ref[...] = jnp.zeros((_G, _H), jnp.float32)
        cnt_ref[...] = jnp.zeros((_G, 1), jnp.float32)

    pooled_ref[...] += jnp.dot(oh, havg, preferred_element_type=jnp.float32)
    cnt_ref[...] += jnp.sum(oh, axis=1, keepdims=True)


_pool_call = pl.pallas_call(
    _pool_body,
    grid=(_NB,),
    in_specs=[
        pl.BlockSpec((_S, _R, _H), lambda i: (0, i, 0)),
        pl.BlockSpec((_S, 1, _H), lambda i: (0, 0, 0)),
        pl.BlockSpec((_S, 1, _H), lambda i: (0, 0, 0)),
        pl.BlockSpec((_S, 1, _H), lambda i: (0, 0, 0)),
        pl.BlockSpec((_S, 1, _H), lambda i: (0, 0, 0)),
        pl.BlockSpec((1, 1, _R), lambda i: (i, 0, 0)),
    ],
    out_specs=[
        pl.BlockSpec((_G, _H), lambda i: (0, 0)),
        pl.BlockSpec((_G, 1), lambda i: (0, 0)),
    ],
    out_shape=[
        jax.ShapeDtypeStruct((_G, _H), jnp.float32),
        jax.ShapeDtypeStruct((_G, 1), jnp.float32),
    ],
)


def _head_body(pooled_ref, cnt_ref, fc1w_ref, fc1b_ref, outw_ref, outb_ref,
               out_ref):
    y = pooled_ref[...] / jnp.maximum(cnt_ref[...], 1.0)
    nrm = jnp.sqrt(jnp.sum(y * y, axis=1, keepdims=True))
    h = y / jnp.maximum(nrm, 1e-12)
    h1 = lax.dot_general(h, fc1w_ref[...], _NT,
                         preferred_element_type=jnp.float32) + fc1b_ref[...]
    h1 = jnp.where(h1 >= 0, h1, 0.01 * h1)
    o = jnp.sum(h1 * outw_ref[...], axis=1, keepdims=True) + outb_ref[...]
    out_ref[...] = 1.0 / (1.0 + jnp.exp(-o))


_head_call = pl.pallas_call(
    _head_body,
    out_shape=jax.ShapeDtypeStruct((_G, 1), jnp.float32),
)


def kernel(x, fc_proj_W, gcn_W, bn_gamma, bn_beta, fc1_W, fc1_b, out_W,
           out_b, edge_index, batch):
    src = edge_index[0].reshape(_NS, _NG, _CHG, _CHP)
    dst = edge_index[1].reshape(_NS, _NG, _CHG, _CHP)

    deg2 = _sc_offload(_deg_kernel)(edge_index[1].reshape(_NW, _NCHUNK, _CH))
    xn, normb = _prep_call(deg2, x)

    y2 = _sc_offload(_pass_kernel)(src, dst, xn)
    z = _combine_call(y2, normb)

    t, ssum, ssq = _b1_call(z, fc_proj_W, gcn_W[:, 0])
    hn = _b2_call(t, ssum, ssq,
                  bn_gamma[:, 0].reshape(_S, 1, _H),
                  bn_beta[:, 0].reshape(_S, 1, _H), normb)

    agg5 = _sc_offload(_pass5_kernel)(src, dst, hn)
    u, csum, csq = _c1_call(agg5, normb, gcn_W[:, 1])
    pooled, cnt = _pool_call(u, csum, csq,
                             bn_gamma[:, 1].reshape(_S, 1, _H),
                             bn_beta[:, 1].reshape(_S, 1, _H),
                             batch.reshape(_NB, 1, _R))
    return _head_call(pooled, cnt, fc1_W, fc1_b.reshape(1, _H), out_W,
                      out_b.reshape(1, 1))
